# Initial kernel scaffold; baseline (speedup 1.0000x reference)
import jax
import jax.numpy as jnp
from jax.experimental import pallas as pl


def kernel(kpts_A, kpts_B, kpts_A_to_B, kpts_B_to_A, descriptions_A, descriptions_B):
    raise NotImplementedError("write your pallas kernel here")


# speedup vs baseline: 1.7002x; 1.7002x over previous
import functools

import jax
import jax.numpy as jnp
from jax.experimental import pallas as pl
from jax.experimental.pallas import tpu as pltpu

B, N, D = 8, 2048, 256
CHUNK = 512
NCHUNK = N // CHUNK


def _dist_chunk(ax, ay, bx, by):
    dx = ax - bx
    dy = ay - by
    return jnp.sqrt(jnp.maximum(dx * dx + dy * dy, 0.0))


def _body(kA_ref, kBt_ref, kAB_ref, kBAt_ref, dA_ref, dB_ref,
          out_ref, corr_scr):
    a = dA_ref[0]
    b = dB_ref[0]
    na = a / jnp.sqrt(jnp.sum(a * a, axis=-1, keepdims=True))
    nb = b / jnp.sqrt(jnp.sum(b * b, axis=-1, keepdims=True))

    kB_x = kBt_ref[0, 0:1, :]
    kB_y = kBt_ref[0, 1:2, :]
    kBA_x = kBAt_ref[0, 0:1, :]
    kBA_y = kBAt_ref[0, 1:2, :]

    se_c = jnp.zeros((1, N), jnp.float32)
    min_A = jnp.full((1, N), jnp.inf, jnp.float32)
    se_r_chunks = []
    min_B_chunks = []
    for ci in range(NCHUNK):
        r0 = ci * CHUNK
        c = jax.lax.dot_general(
            na[r0:r0 + CHUNK], nb,
            (((1,), (1,)), ((), ())),
            preferred_element_type=jnp.float32) * 20.0
        corr_scr[r0:r0 + CHUNK, :] = c
        e = jnp.exp(c)
        se_r_chunks.append(jnp.sum(e, axis=1, keepdims=True))
        se_c = se_c + jnp.sum(e, axis=0, keepdims=True)
        d_B = _dist_chunk(kAB_ref[0, r0:r0 + CHUNK, 0:1],
                          kAB_ref[0, r0:r0 + CHUNK, 1:2], kB_x, kB_y)
        d_A = _dist_chunk(kA_ref[0, r0:r0 + CHUNK, 0:1],
                          kA_ref[0, r0:r0 + CHUNK, 1:2], kBA_x, kBA_y)
        min_B_chunks.append(jnp.min(d_B, axis=1, keepdims=True))
        min_A = jnp.minimum(min_A, jnp.min(d_A, axis=0, keepdims=True))

    lse_c = jnp.log(se_c)

    num = jnp.float32(0.0)
    cnt = jnp.float32(0.0)
    for ci in range(NCHUNK):
        r0 = ci * CHUNK
        c = corr_scr[r0:r0 + CHUNK, :]
        d_B = _dist_chunk(kAB_ref[0, r0:r0 + CHUNK, 0:1],
                          kAB_ref[0, r0:r0 + CHUNK, 1:2], kB_x, kB_y)
        d_A = _dist_chunk(kA_ref[0, r0:r0 + CHUNK, 0:1],
                          kA_ref[0, r0:r0 + CHUNK, 1:2], kBA_x, kBA_y)
        mask = ((d_B == min_B_chunks[ci]) & (d_A == min_A)
                & (d_B < 0.01) & (d_A < 0.01))
        logp = (2.0 * c - jnp.log(se_r_chunks[ci])) - lse_c
        num = num + jnp.sum(jnp.where(mask, logp, 0.0))
        cnt = cnt + jnp.sum(mask.astype(jnp.float32))

    lane = jax.lax.broadcasted_iota(jnp.int32, (1, 1, 128), 2)
    out_ref[...] = jnp.where(lane == 0, num, cnt)


@jax.jit
def kernel(kpts_A, kpts_B, kpts_A_to_B, kpts_B_to_A,
           descriptions_A, descriptions_B):
    kBt = jnp.swapaxes(kpts_B, 1, 2)
    kBAt = jnp.swapaxes(kpts_B_to_A, 1, 2)

    batch_spec = lambda shp: pl.BlockSpec((1,) + shp, lambda i: (i, 0, 0))
    out = pl.pallas_call(
        _body,
        grid=(B,),
        in_specs=[
            batch_spec((N, 2)),
            batch_spec((2, N)),
            batch_spec((N, 2)),
            batch_spec((2, N)),
            batch_spec((N, D)),
            batch_spec((N, D)),
        ],
        out_specs=pl.BlockSpec((1, 1, 128), lambda i: (i, 0, 0)),
        out_shape=jax.ShapeDtypeStruct((B, 1, 128), jnp.float32),
        scratch_shapes=[pltpu.VMEM((N, N), jnp.float32)],
    )(kpts_A, kBt, kpts_A_to_B, kBAt, descriptions_A, descriptions_B)

    total_num = jnp.sum(out[:, 0, 0])
    total_cnt = jnp.sum(out[:, 0, 1])
    return -total_num / jnp.maximum(total_cnt, 1.0)


# speedup vs baseline: 3.7150x; 2.1851x over previous
import jax
import jax.numpy as jnp
from jax.experimental import pallas as pl
from jax.experimental.pallas import tpu as pltpu

B, N, D = 8, 2048, 256
CHUNK = 512
NCHUNK = N // CHUNK
BIG = 1 << 30
THRESH2 = 1e-4


def _body(fAB_ref, fBt_ref, fA_ref, fBAt_ref, dA_ref, dB_ref, out_ref):
    a = dA_ref[0]
    b = dB_ref[0]
    na = a * (20.0 / jnp.sqrt(jnp.sum(a * a, axis=-1, keepdims=True)))
    nb = b / jnp.sqrt(jnp.sum(b * b, axis=-1, keepdims=True))

    fBt = fBt_ref[0]
    fBAt = fBAt_ref[0]

    se_c = jnp.zeros((1, N), jnp.float32)
    min_A = jnp.full((1, N), jnp.inf, jnp.float32)
    arg_A = jnp.full((1, N), BIG, jnp.int32)
    se_r_chunks = []
    min_B_chunks = []
    jstar_chunks = []
    corrsel_chunks = []
    for ci in range(NCHUNK):
        r0 = ci * CHUNK
        corr = jax.lax.dot_general(
            na[r0:r0 + CHUNK], nb, (((1,), (1,)), ((), ())),
            preferred_element_type=jnp.float32)
        e = jnp.exp(corr)
        se_r_chunks.append(jnp.sum(e, axis=1, keepdims=True))
        se_c = se_c + jnp.sum(e, axis=0, keepdims=True)

        iota_m = jax.lax.broadcasted_iota(jnp.int32, (CHUNK, N), 1)
        iota_n = jax.lax.broadcasted_iota(jnp.int32, (CHUNK, N), 0) + r0

        d2B = jnp.dot(fAB_ref[0, r0:r0 + CHUNK, :], fBt,
                      preferred_element_type=jnp.float32)
        mB = jnp.min(d2B, axis=1, keepdims=True)
        min_B_chunks.append(mB)
        js = jnp.min(jnp.where(d2B == mB, iota_m, BIG), axis=1,
                     keepdims=True)
        jstar_chunks.append(js)
        corrsel_chunks.append(jnp.max(
            jnp.where(iota_m == js, corr, -jnp.inf), axis=1, keepdims=True))

        d2A = jnp.dot(fA_ref[0, r0:r0 + CHUNK, :], fBAt,
                      preferred_element_type=jnp.float32)
        cmin = jnp.min(d2A, axis=0, keepdims=True)
        carg = jnp.min(jnp.where(d2A == cmin, iota_n, BIG), axis=0,
                       keepdims=True)
        upd = cmin < min_A
        arg_A = jnp.where(upd, carg, arg_A)
        min_A = jnp.minimum(min_A, cmin)

    lse_c = jnp.log(se_c)

    num = jnp.float32(0.0)
    cnt = jnp.float32(0.0)
    for ci in range(NCHUNK):
        r0 = ci * CHUNK
        iota_m = jax.lax.broadcasted_iota(jnp.int32, (CHUNK, N), 1)
        iota_n = jax.lax.broadcasted_iota(jnp.int32, (CHUNK, N), 0) + r0
        sel = (iota_m == jstar_chunks[ci]) & (arg_A == iota_n)
        ok = (sel & (min_B_chunks[ci] < THRESH2) & (min_A < THRESH2))
        lse_r = jnp.log(se_r_chunks[ci])
        val = (2.0 * corrsel_chunks[ci] - lse_r) - lse_c
        num = num + jnp.sum(jnp.where(ok, val, 0.0))
        cnt = cnt + jnp.sum(ok.astype(jnp.float32))

    lane = jax.lax.broadcasted_iota(jnp.int32, (1, 1, 128), 2)
    out_ref[...] = jnp.where(lane == 0, num, cnt)


def _row_feats(p):
    n2 = jnp.sum(p * p, axis=-1, keepdims=True)
    ones = jnp.ones_like(n2)
    zer = jnp.zeros((p.shape[0], p.shape[1], 4), p.dtype)
    return jnp.concatenate([n2, -2.0 * p, ones, zer], axis=-1)


def _col_feats(p):
    n2 = jnp.sum(p * p, axis=-1, keepdims=True)
    ones = jnp.ones_like(n2)
    zer = jnp.zeros((p.shape[0], p.shape[1], 4), p.dtype)
    f = jnp.concatenate([ones, p, n2, zer], axis=-1)
    return jnp.swapaxes(f, 1, 2)


@jax.jit
def kernel(kpts_A, kpts_B, kpts_A_to_B, kpts_B_to_A,
           descriptions_A, descriptions_B):
    fAB = _row_feats(kpts_A_to_B)
    fBt = _col_feats(kpts_B)
    fA = _row_feats(kpts_A)
    fBAt = _col_feats(kpts_B_to_A)

    batch_spec = lambda shp: pl.BlockSpec((1,) + shp, lambda i: (i, 0, 0))
    out = pl.pallas_call(
        _body,
        grid=(B,),
        in_specs=[
            batch_spec((N, 8)),
            batch_spec((8, N)),
            batch_spec((N, 8)),
            batch_spec((8, N)),
            batch_spec((N, D)),
            batch_spec((N, D)),
        ],
        out_specs=pl.BlockSpec((1, 1, 128), lambda i: (i, 0, 0)),
        out_shape=jax.ShapeDtypeStruct((B, 1, 128), jnp.float32),
    )(fAB, fBt, fA, fBAt, descriptions_A, descriptions_B)

    total_num = jnp.sum(out[:, 0, 0])
    total_cnt = jnp.sum(out[:, 0, 1])
    return -total_num / jnp.maximum(total_cnt, 1.0)
